# baseline (speedup 1.0000x reference)
import functools

import jax
import jax.numpy as jnp
from jax.experimental import pallas as pl
from jax.experimental.pallas import tpu as pltpu

NUM_HEADS = 16
HEAD_DIM = 128


def _cast_kernel(a_ref, b_ref, c_ref, d_ref, oa_ref, ob_ref, oc_ref, od_ref):
    oa_ref[...] = a_ref[...].astype(jnp.bfloat16)
    ob_ref[...] = b_ref[...].astype(jnp.bfloat16)
    oc_ref[...] = c_ref[...].astype(jnp.bfloat16)
    od_ref[...] = d_ref[...].astype(jnp.bfloat16)


def _cast_weights(wq, wk, wv, wo, interpret=False):
    n, k = wq.shape
    bm = 256
    spec = pl.BlockSpec((bm, k), lambda i: (i, 0))
    out = jax.ShapeDtypeStruct((n, k), jnp.bfloat16)
    return pl.pallas_call(
        _cast_kernel, grid=(n // bm,),
        in_specs=[spec] * 4, out_specs=[spec] * 4,
        out_shape=[out] * 4, interpret=interpret)(wq, wk, wv, wo)


def _qkv_kernel(x_ref, wq_ref, wk_ref, wv_ref, q_ref, k_ref, v_ref, *, scale):
    xb = x_ref[...].astype(jnp.bfloat16)
    dn = (((1,), (1,)), ((), ()))
    q_acc = jax.lax.dot_general(
        xb, wq_ref[...], dn, preferred_element_type=jnp.float32)
    q_ref[...] = (q_acc * scale).astype(jnp.bfloat16)
    k_ref[...] = jax.lax.dot_general(
        xb, wk_ref[...], dn, preferred_element_type=jnp.float32
    ).astype(jnp.bfloat16)
    v_ref[...] = jax.lax.dot_general(
        xb, wv_ref[...], dn, preferred_element_type=jnp.float32
    ).astype(jnp.bfloat16)


def _qkv_proj(x2, wqb, wkb, wvb, bm, interpret=False):
    m, k = x2.shape
    n = wqb.shape[0]
    scale = 1.0 / (HEAD_DIM ** 0.5)
    x_spec = pl.BlockSpec((bm, k), lambda i: (i, 0))
    w_spec = pl.BlockSpec((n, k), lambda i: (0, 0))
    o_spec = pl.BlockSpec((bm, n), lambda i: (i, 0))
    out = jax.ShapeDtypeStruct((m, n), jnp.bfloat16)
    return pl.pallas_call(
        functools.partial(_qkv_kernel, scale=scale), grid=(m // bm,),
        in_specs=[x_spec, w_spec, w_spec, w_spec],
        out_specs=[o_spec] * 3,
        out_shape=[out] * 3, interpret=interpret)(x2, wqb, wkb, wvb)


def _out_kernel(a_ref, w_ref, b_ref, o_ref):
    acc = jax.lax.dot_general(
        a_ref[...], w_ref[...], (((1,), (1,)), ((), ())),
        preferred_element_type=jnp.float32)
    o_ref[...] = acc + b_ref[...]


def _out_proj(attn2, wob, bo, bm, interpret=False):
    m, k = attn2.shape
    n = wob.shape[0]
    a_spec = pl.BlockSpec((bm, k), lambda i: (i, 0))
    w_spec = pl.BlockSpec((n, k), lambda i: (0, 0))
    b_spec = pl.BlockSpec((1, n), lambda i: (0, 0))
    o_spec = pl.BlockSpec((bm, n), lambda i: (i, 0))
    return pl.pallas_call(
        _out_kernel, grid=(m // bm,),
        in_specs=[a_spec, w_spec, b_spec],
        out_specs=o_spec,
        out_shape=jax.ShapeDtypeStruct((m, n), jnp.float32),
        interpret=interpret)(attn2, wob, bo.reshape(1, n))


def _flash_kernel(q_ref, k_ref, v_ref, o_ref, s_scr, *, bq, bk):
    qi = pl.program_id(1)
    q = q_ref[0]
    nlanes = 128
    ncol = bk // nlanes

    def logits(j):
        kb = k_ref[0, pl.ds(j * bk, bk), :]
        return jax.lax.dot_general(
            q, kb, dimension_numbers=(((1,), (1,)), ((), ())),
            preferred_element_type=jnp.float32)

    def fold_max(macc, s):
        return jnp.maximum(macc, jnp.max(s.reshape(bq, ncol, nlanes), axis=1))

    def pass_a(j, macc):
        s = logits(j)
        s_scr[:, pl.ds(j * bk, bk)] = s
        return fold_max(macc, s)

    macc = jax.lax.fori_loop(
        0, qi, pass_a, jnp.full((bq, nlanes), -jnp.inf, jnp.float32))
    s = logits(qi)
    rows = jax.lax.broadcasted_iota(jnp.int32, (bq, bk), 0)
    cols = jax.lax.broadcasted_iota(jnp.int32, (bq, bk), 1)
    s = jnp.where(cols <= rows, s, -jnp.inf)
    s_scr[:, pl.ds(qi * bk, bk)] = s
    macc = fold_max(macc, s)
    m = jnp.max(macc, axis=1, keepdims=True)

    ones = jnp.ones((bk, HEAD_DIM), jnp.bfloat16)

    def pass_b(j, carry):
        lacc, acc = carry
        p = jnp.exp(s_scr[:, pl.ds(j * bk, bk)] - m).astype(jnp.bfloat16)
        lacc = lacc + jnp.dot(p, ones, preferred_element_type=jnp.float32)
        vb = v_ref[0, pl.ds(j * bk, bk), :]
        acc = acc + jnp.dot(p, vb, preferred_element_type=jnp.float32)
        return lacc, acc

    lacc, acc = jax.lax.fori_loop(
        0, qi + 1, pass_b, (jnp.zeros((bq, HEAD_DIM), jnp.float32),
                            jnp.zeros((bq, HEAD_DIM), jnp.float32)))
    o_ref[0] = (acc / lacc).astype(o_ref.dtype)


def _flash_attention(q, k, v, bq, bk, interpret=False):
    b, s, hidden = q.shape
    grid = (b * NUM_HEADS, s // bq)
    q_spec = pl.BlockSpec(
        (1, bq, HEAD_DIM),
        lambda bh, qi: (bh // NUM_HEADS, qi, bh % NUM_HEADS))
    kv_spec = pl.BlockSpec(
        (1, s, HEAD_DIM),
        lambda bh, qi: (bh // NUM_HEADS, 0, bh % NUM_HEADS))
    o_spec = pl.BlockSpec(
        (1, bq, HEAD_DIM),
        lambda bh, qi: (bh // NUM_HEADS, qi, bh % NUM_HEADS))
    return pl.pallas_call(
        functools.partial(_flash_kernel, bq=bq, bk=bk),
        grid=grid,
        in_specs=[q_spec, kv_spec, kv_spec],
        out_specs=o_spec,
        out_shape=jax.ShapeDtypeStruct((b, s, hidden), jnp.bfloat16),
        scratch_shapes=[pltpu.VMEM((bq, s), jnp.float32)],
        interpret=interpret)(q, k, v)


def kernel(x, Wq, Wk, Wv, Wo, bo, interpret=False):
    b, s, hidden = x.shape
    wqb, wkb, wvb, wob = _cast_weights(Wq, Wk, Wv, Wo, interpret=interpret)
    x2 = x.reshape(b * s, hidden)
    q2, k2, v2 = _qkv_proj(x2, wqb, wkb, wvb, bm=512, interpret=interpret)
    q3 = q2.reshape(b, s, hidden)
    k3 = k2.reshape(b, s, hidden)
    v3 = v2.reshape(b, s, hidden)
    attn = _flash_attention(q3, k3, v3, bq=512, bk=512, interpret=interpret)
    out = _out_proj(attn.reshape(b * s, hidden), wob, bo, bm=512,
                    interpret=interpret)
    return out.reshape(b, s, hidden)


# speedup vs baseline: 1.6079x; 1.6079x over previous
import functools

import jax
import jax.numpy as jnp
from jax.experimental import pallas as pl
from jax.experimental.pallas import tpu as pltpu

NUM_HEADS = 16
HEAD_DIM = 128


def _cast_kernel(a_ref, b_ref, c_ref, d_ref, oa_ref, ob_ref, oc_ref, od_ref):
    oa_ref[...] = a_ref[...].astype(jnp.bfloat16)
    ob_ref[...] = b_ref[...].astype(jnp.bfloat16)
    oc_ref[...] = c_ref[...].astype(jnp.bfloat16)
    od_ref[...] = d_ref[...].astype(jnp.bfloat16)


def _cast_weights(wq, wk, wv, wo, interpret=False):
    n, k = wq.shape
    bm = 256
    spec = pl.BlockSpec((bm, k), lambda i: (i, 0))
    out = jax.ShapeDtypeStruct((n, k), jnp.bfloat16)
    return pl.pallas_call(
        _cast_kernel, grid=(n // bm,),
        in_specs=[spec] * 4, out_specs=[spec] * 4,
        out_shape=[out] * 4, interpret=interpret)(wq, wk, wv, wo)


def _qkv_kernel(x_ref, wq_ref, wk_ref, wv_ref, q_ref, k_ref, v_ref, *, scale):
    xb = x_ref[...].astype(jnp.bfloat16)
    dn = (((1,), (1,)), ((), ()))
    q_acc = jax.lax.dot_general(
        xb, wq_ref[...], dn, preferred_element_type=jnp.float32)
    q_ref[...] = (q_acc * scale).astype(jnp.bfloat16)
    k_ref[...] = jax.lax.dot_general(
        xb, wk_ref[...], dn, preferred_element_type=jnp.float32
    ).astype(jnp.bfloat16)
    v_ref[...] = jax.lax.dot_general(
        xb, wv_ref[...], dn, preferred_element_type=jnp.float32
    ).astype(jnp.bfloat16)


def _qkv_proj(x2, wqb, wkb, wvb, bm, interpret=False):
    m, k = x2.shape
    n = wqb.shape[0]
    scale = 1.0 / (HEAD_DIM ** 0.5)
    x_spec = pl.BlockSpec((bm, k), lambda i: (i, 0))
    w_spec = pl.BlockSpec((n, k), lambda i: (0, 0))
    o_spec = pl.BlockSpec((bm, n), lambda i: (i, 0))
    out = jax.ShapeDtypeStruct((m, n), jnp.bfloat16)
    return pl.pallas_call(
        functools.partial(_qkv_kernel, scale=scale), grid=(m // bm,),
        in_specs=[x_spec, w_spec, w_spec, w_spec],
        out_specs=[o_spec] * 3,
        out_shape=[out] * 3, interpret=interpret)(x2, wqb, wkb, wvb)


def _out_kernel(a_ref, w_ref, b_ref, o_ref):
    acc = jax.lax.dot_general(
        a_ref[...], w_ref[...], (((1,), (1,)), ((), ())),
        preferred_element_type=jnp.float32)
    o_ref[...] = acc + b_ref[...]


def _out_proj(attn2, wob, bo, bm, interpret=False):
    m, k = attn2.shape
    n = wob.shape[0]
    a_spec = pl.BlockSpec((bm, k), lambda i: (i, 0))
    w_spec = pl.BlockSpec((n, k), lambda i: (0, 0))
    b_spec = pl.BlockSpec((1, n), lambda i: (0, 0))
    o_spec = pl.BlockSpec((bm, n), lambda i: (i, 0))
    return pl.pallas_call(
        _out_kernel, grid=(m // bm,),
        in_specs=[a_spec, w_spec, b_spec],
        out_specs=o_spec,
        out_shape=jax.ShapeDtypeStruct((m, n), jnp.float32),
        interpret=interpret)(attn2, wob, bo.reshape(1, n))


def _flash_kernel(q_ref, k_ref, v_ref, o_ref, s_scr, *, bq, bk):
    qi = pl.program_id(1)
    q = q_ref[0]
    nlanes = 128
    ncol = bk // nlanes

    def logits(j):
        kb = k_ref[0, pl.ds(j * bk, bk), :]
        return jax.lax.dot_general(
            q, kb, dimension_numbers=(((1,), (1,)), ((), ())),
            preferred_element_type=jnp.float32)

    def fold_max(macc, s):
        for c in range(ncol):
            macc = jnp.maximum(
                macc, jax.lax.slice(s, (0, c * nlanes), (bq, (c + 1) * nlanes)))
        return macc

    def pass_a(j, macc):
        s = logits(j)
        s_scr[:, pl.ds(j * bk, bk)] = s
        return fold_max(macc, s)

    macc = jax.lax.fori_loop(
        0, qi, pass_a, jnp.full((bq, nlanes), -jnp.inf, jnp.float32))
    s = logits(qi)
    rows = jax.lax.broadcasted_iota(jnp.int32, (bq, bk), 0)
    cols = jax.lax.broadcasted_iota(jnp.int32, (bq, bk), 1)
    s = jnp.where(cols <= rows, s, -jnp.inf)
    s_scr[:, pl.ds(qi * bk, bk)] = s
    macc = fold_max(macc, s)
    m = jnp.max(macc, axis=1, keepdims=True)

    ones = jnp.ones((bk, HEAD_DIM), jnp.bfloat16)

    def pass_b(j, carry):
        lacc, acc = carry
        p = jnp.exp(s_scr[:, pl.ds(j * bk, bk)] - m).astype(jnp.bfloat16)
        lacc = lacc + jnp.dot(p, ones, preferred_element_type=jnp.float32)
        vb = v_ref[0, pl.ds(j * bk, bk), :]
        acc = acc + jnp.dot(p, vb, preferred_element_type=jnp.float32)
        return lacc, acc

    lacc, acc = jax.lax.fori_loop(
        0, qi + 1, pass_b, (jnp.zeros((bq, HEAD_DIM), jnp.float32),
                            jnp.zeros((bq, HEAD_DIM), jnp.float32)))
    o_ref[0] = (acc / lacc).astype(o_ref.dtype)


def _flash_attention(q, k, v, bq, bk, interpret=False):
    b, s, hidden = q.shape
    grid = (b * NUM_HEADS, s // bq)
    q_spec = pl.BlockSpec(
        (1, bq, HEAD_DIM),
        lambda bh, qi: (bh // NUM_HEADS, qi, bh % NUM_HEADS))
    kv_spec = pl.BlockSpec(
        (1, s, HEAD_DIM),
        lambda bh, qi: (bh // NUM_HEADS, 0, bh % NUM_HEADS))
    o_spec = pl.BlockSpec(
        (1, bq, HEAD_DIM),
        lambda bh, qi: (bh // NUM_HEADS, qi, bh % NUM_HEADS))
    return pl.pallas_call(
        functools.partial(_flash_kernel, bq=bq, bk=bk),
        grid=grid,
        in_specs=[q_spec, kv_spec, kv_spec],
        out_specs=o_spec,
        out_shape=jax.ShapeDtypeStruct((b, s, hidden), jnp.bfloat16),
        scratch_shapes=[pltpu.VMEM((bq, s), jnp.float32)],
        interpret=interpret)(q, k, v)


def kernel(x, Wq, Wk, Wv, Wo, bo, interpret=False):
    b, s, hidden = x.shape
    wqb, wkb, wvb, wob = _cast_weights(Wq, Wk, Wv, Wo, interpret=interpret)
    x2 = x.reshape(b * s, hidden)
    q2, k2, v2 = _qkv_proj(x2, wqb, wkb, wvb, bm=512, interpret=interpret)
    q3 = q2.reshape(b, s, hidden)
    k3 = k2.reshape(b, s, hidden)
    v3 = v2.reshape(b, s, hidden)
    attn = _flash_attention(q3, k3, v3, bq=512, bk=512, interpret=interpret)
    out = _out_proj(attn.reshape(b * s, hidden), wob, bo, bm=512,
                    interpret=interpret)
    return out.reshape(b, s, hidden)


# speedup vs baseline: 2.1143x; 1.3149x over previous
import functools

import jax
import jax.numpy as jnp
from jax.experimental import pallas as pl
from jax.experimental.pallas import tpu as pltpu

NUM_HEADS = 16
HEAD_DIM = 128


def _cast_kernel(a_ref, b_ref, c_ref, d_ref, oa_ref, ob_ref, oc_ref, od_ref):
    oa_ref[...] = a_ref[...].astype(jnp.bfloat16)
    ob_ref[...] = b_ref[...].astype(jnp.bfloat16)
    oc_ref[...] = c_ref[...].astype(jnp.bfloat16)
    od_ref[...] = d_ref[...].astype(jnp.bfloat16)


def _cast_weights(wq, wk, wv, wo, interpret=False):
    n, k = wq.shape
    bm = 256
    spec = pl.BlockSpec((bm, k), lambda i: (i, 0))
    out = jax.ShapeDtypeStruct((n, k), jnp.bfloat16)
    return pl.pallas_call(
        _cast_kernel, grid=(n // bm,),
        in_specs=[spec] * 4, out_specs=[spec] * 4,
        out_shape=[out] * 4, interpret=interpret)(wq, wk, wv, wo)


def _qkv_kernel(x_ref, wq_ref, wk_ref, wv_ref, q_ref, k_ref, v_ref, *, scale):
    xb = x_ref[...].astype(jnp.bfloat16)
    dn = (((1,), (1,)), ((), ()))
    q_acc = jax.lax.dot_general(
        xb, wq_ref[...], dn, preferred_element_type=jnp.float32)
    q_ref[...] = (q_acc * scale).astype(jnp.bfloat16)
    k_ref[...] = jax.lax.dot_general(
        xb, wk_ref[...], dn, preferred_element_type=jnp.float32
    ).astype(jnp.bfloat16)
    v_ref[...] = jax.lax.dot_general(
        xb, wv_ref[...], dn, preferred_element_type=jnp.float32
    ).astype(jnp.bfloat16)


def _qkv_proj(x2, wqb, wkb, wvb, bm, interpret=False):
    m, k = x2.shape
    n = wqb.shape[0]
    scale = 1.0 / (HEAD_DIM ** 0.5)
    x_spec = pl.BlockSpec((bm, k), lambda i: (i, 0))
    w_spec = pl.BlockSpec((n, k), lambda i: (0, 0))
    o_spec = pl.BlockSpec((bm, n), lambda i: (i, 0))
    out = jax.ShapeDtypeStruct((m, n), jnp.bfloat16)
    return pl.pallas_call(
        functools.partial(_qkv_kernel, scale=scale), grid=(m // bm,),
        in_specs=[x_spec, w_spec, w_spec, w_spec],
        out_specs=[o_spec] * 3,
        out_shape=[out] * 3, interpret=interpret)(x2, wqb, wkb, wvb)


def _out_kernel(a_ref, w_ref, b_ref, o_ref):
    acc = jax.lax.dot_general(
        a_ref[...], w_ref[...], (((1,), (1,)), ((), ())),
        preferred_element_type=jnp.float32)
    o_ref[...] = acc + b_ref[...]


def _out_proj(attn2, wob, bo, bm, interpret=False):
    m, k = attn2.shape
    n = wob.shape[0]
    a_spec = pl.BlockSpec((bm, k), lambda i: (i, 0))
    w_spec = pl.BlockSpec((n, k), lambda i: (0, 0))
    b_spec = pl.BlockSpec((1, n), lambda i: (0, 0))
    o_spec = pl.BlockSpec((bm, n), lambda i: (i, 0))
    return pl.pallas_call(
        _out_kernel, grid=(m // bm,),
        in_specs=[a_spec, w_spec, b_spec],
        out_specs=o_spec,
        out_shape=jax.ShapeDtypeStruct((m, n), jnp.float32),
        interpret=interpret)(attn2, wob, bo.reshape(1, n))


def _flash_kernel(q_ref, k_ref, v_ref, o_ref, s_scr, *, bq, bk, nq):
    qi = pl.program_id(1)
    q = q_ref[0]
    nlanes = 128
    ncol = bk // nlanes
    rows = jax.lax.broadcasted_iota(jnp.int32, (bq, bk), 0)
    cols = jax.lax.broadcasted_iota(jnp.int32, (bq, bk), 1)
    ones = jnp.ones((bk, HEAD_DIM), jnp.bfloat16)

    def fold_max(macc, s):
        for c in range(ncol):
            macc = jnp.maximum(
                macc, jax.lax.slice(s, (0, c * nlanes), (bq, (c + 1) * nlanes)))
        return macc

    for sqi in range(nq):
        @pl.when(qi == sqi)
        def _(sqi=sqi):
            macc = jnp.full((bq, nlanes), -jnp.inf, jnp.float32)
            for j in range(sqi + 1):
                kb = k_ref[0, j * bk:(j + 1) * bk, :]
                s = jax.lax.dot_general(
                    q, kb, dimension_numbers=(((1,), (1,)), ((), ())),
                    preferred_element_type=jnp.float32)
                if j == sqi:
                    s = jnp.where(cols <= rows, s, -jnp.inf)
                s_scr[:, j * bk:(j + 1) * bk] = s
                macc = fold_max(macc, s)
            m = jnp.max(macc, axis=1, keepdims=True)
            lacc = jnp.zeros((bq, HEAD_DIM), jnp.float32)
            acc = jnp.zeros((bq, HEAD_DIM), jnp.float32)
            for j in range(sqi + 1):
                p = jnp.exp(s_scr[:, j * bk:(j + 1) * bk] - m
                            ).astype(jnp.bfloat16)
                lacc = lacc + jnp.dot(p, ones,
                                      preferred_element_type=jnp.float32)
                vb = v_ref[0, j * bk:(j + 1) * bk, :]
                acc = acc + jnp.dot(p, vb, preferred_element_type=jnp.float32)
            o_ref[0] = (acc / lacc).astype(o_ref.dtype)


def _flash_attention(q, k, v, bq, bk, interpret=False):
    b, s, hidden = q.shape
    grid = (b * NUM_HEADS, s // bq)
    q_spec = pl.BlockSpec(
        (1, bq, HEAD_DIM),
        lambda bh, qi: (bh // NUM_HEADS, qi, bh % NUM_HEADS))
    kv_spec = pl.BlockSpec(
        (1, s, HEAD_DIM),
        lambda bh, qi: (bh // NUM_HEADS, 0, bh % NUM_HEADS))
    o_spec = pl.BlockSpec(
        (1, bq, HEAD_DIM),
        lambda bh, qi: (bh // NUM_HEADS, qi, bh % NUM_HEADS))
    return pl.pallas_call(
        functools.partial(_flash_kernel, bq=bq, bk=bk, nq=s // bq),
        grid=grid,
        in_specs=[q_spec, kv_spec, kv_spec],
        out_specs=o_spec,
        out_shape=jax.ShapeDtypeStruct((b, s, hidden), jnp.bfloat16),
        scratch_shapes=[pltpu.VMEM((bq, s), jnp.float32)],
        interpret=interpret)(q, k, v)


def kernel(x, Wq, Wk, Wv, Wo, bo, interpret=False):
    b, s, hidden = x.shape
    wqb, wkb, wvb, wob = _cast_weights(Wq, Wk, Wv, Wo, interpret=interpret)
    x2 = x.reshape(b * s, hidden)
    q2, k2, v2 = _qkv_proj(x2, wqb, wkb, wvb, bm=512, interpret=interpret)
    q3 = q2.reshape(b, s, hidden)
    k3 = k2.reshape(b, s, hidden)
    v3 = v2.reshape(b, s, hidden)
    attn = _flash_attention(q3, k3, v3, bq=512, bk=512, interpret=interpret)
    out = _out_proj(attn.reshape(b * s, hidden), wob, bo, bm=512,
                    interpret=interpret)
    return out.reshape(b, s, hidden)
